# Initial kernel scaffold; baseline (speedup 1.0000x reference)
import functools

import jax
import jax.numpy as jnp
from jax import lax
from jax.experimental import pallas as pl
from jax.experimental.pallas import tpu as pltpu
from jax.experimental.pallas import tpu_sc as plsc

B = 16384
BAG = 50
D = 64
V = 1000000

NC = 2
NS = 16
NW = NC * NS

BAGS_PER_W = B // NW
ROWS_PER_W = BAGS_PER_W * BAG
CHUNK_BAGS = 8
CHUNK_ROWS = CHUNK_BAGS * BAG
N_CHUNKS = BAGS_PER_W // CHUNK_BAGS
SUB = 80
N_SUB = CHUNK_ROWS // SUB


def _fire(w_hbm, idx_v, buf, sem, g):
    base = g * CHUNK_ROWS
    for s in range(N_SUB):
        pltpu.async_copy(
            w_hbm.at[idx_v.at[pl.ds(base + s * SUB, SUB)]],
            buf.at[pl.ds(s * SUB, SUB)],
            sem,
        )


def _drain(w_hbm, buf, sem):
    pltpu.make_async_copy(w_hbm.at[pl.ds(0, CHUNK_ROWS)], buf, sem).wait()


def _accumulate(buf, out_v, bias_vecs, g):
    for bb in range(CHUNK_BAGS):
        row0 = bb * BAG

        def body(j, accs):
            r = row0 + j
            return tuple(
                accs[k] + buf[r, pl.ds(16 * k, 16)] for k in range(4)
            )

        accs = lax.fori_loop(0, BAG, body, bias_vecs, unroll=2)
        orow = g * CHUNK_BAGS + bb
        for k in range(4):
            out_v[orow, pl.ds(16 * k, 16)] = accs[k]


def _sc_body(idx_hbm, w_hbm, bias_hbm, out_hbm,
             idx_v, rows0, rows1, out_v, bias_v, sem0, sem1):
    wid = lax.axis_index("s") * NC + lax.axis_index("c")

    pltpu.sync_copy(bias_hbm, bias_v)
    pltpu.sync_copy(idx_hbm.at[pl.ds(wid * ROWS_PER_W, ROWS_PER_W)], idx_v)

    bias_vecs = tuple(bias_v[pl.ds(16 * k, 16)] for k in range(4))
    bufs = (rows0, rows1)
    sems = (sem0, sem1)

    _fire(w_hbm, idx_v, rows0, sem0, 0)

    def chunk_pair(i, carry):
        for b in range(2):
            g = 2 * i + b
            _fire(w_hbm, idx_v, bufs[1 - b], sems[1 - b], g + 1)
            _drain(w_hbm, bufs[b], sems[b])
            _accumulate(bufs[b], out_v, bias_vecs, g)
        return carry

    lax.fori_loop(0, (N_CHUNKS - 2) // 2, chunk_pair, 0)
    g = N_CHUNKS - 2
    _fire(w_hbm, idx_v, bufs[1], sems[1], g + 1)
    _drain(w_hbm, bufs[0], sems[0])
    _accumulate(bufs[0], out_v, bias_vecs, g)
    _drain(w_hbm, bufs[1], sems[1])
    _accumulate(bufs[1], out_v, bias_vecs, g + 1)

    pltpu.sync_copy(out_v, out_hbm.at[pl.ds(wid * BAGS_PER_W, BAGS_PER_W)])


@jax.jit
def _embed_bag(indices, w, bias):
    mesh = plsc.VectorSubcoreMesh(core_axis_name="c", subcore_axis_name="s")
    run = pl.kernel(
        _sc_body,
        out_type=jax.ShapeDtypeStruct((B, D), jnp.float32),
        mesh=mesh,
        scratch_types=[
            pltpu.VMEM((ROWS_PER_W,), jnp.int32),
            pltpu.VMEM((CHUNK_ROWS, D), jnp.float32),
            pltpu.VMEM((CHUNK_ROWS, D), jnp.float32),
            pltpu.VMEM((BAGS_PER_W, D), jnp.float32),
            pltpu.VMEM((D,), jnp.float32),
            pltpu.SemaphoreType.DMA,
            pltpu.SemaphoreType.DMA,
        ],
    )
    return run(indices, w, bias)


def kernel(indices, offsets, W, bias):
    del offsets
    return _embed_bag(indices.astype(jnp.int32), W, bias)


# speedup vs baseline: 4.2559x; 4.2559x over previous
import functools

import jax
import jax.numpy as jnp
from jax import lax
from jax.experimental import pallas as pl
from jax.experimental.pallas import tpu as pltpu
from jax.experimental.pallas import tpu_sc as plsc

B = 16384
BAG = 50
D = 64
V = 1000000

NC = 2
NS = 16
NW = NC * NS

BAGS_PER_W = B // NW
ROWS_PER_W = BAGS_PER_W * BAG
CHUNK_BAGS = 8
CHUNK_ROWS = CHUNK_BAGS * BAG
N_CHUNKS = BAGS_PER_W // CHUNK_BAGS
SUB = 80
N_SUB = CHUNK_ROWS // SUB


def _fire(w_hbm, idx_v, buf, sem, g):
    base = g * CHUNK_ROWS
    for s in range(N_SUB):
        pltpu.async_copy(
            w_hbm.at[idx_v.at[pl.ds(base + s * SUB, SUB)]],
            buf.at[pl.ds(s * SUB, SUB)],
            sem,
        )


def _drain(w_hbm, buf, sem):
    pltpu.make_async_copy(w_hbm.at[pl.ds(0, CHUNK_ROWS)], buf, sem).wait()


def _accumulate(buf, out_v, bias_vecs, g):
    for bb in range(CHUNK_BAGS):
        row0 = bb * BAG

        def body(j, accs):
            r = row0 + j
            return tuple(
                accs[k] + buf[r, pl.ds(16 * k, 16)] for k in range(4)
            )

        accs = lax.fori_loop(0, BAG, body, bias_vecs, unroll=2)
        orow = g * CHUNK_BAGS + bb
        for k in range(4):
            out_v[orow, pl.ds(16 * k, 16)] = accs[k]


def _sc_body(idx_hbm, w_hbm, bias_hbm, out_hbm,
             idx_v, rows0, rows1, out_v, bias_v, sem0, sem1):
    wid = lax.axis_index("s") * NC + lax.axis_index("c")

    pltpu.sync_copy(bias_hbm, bias_v)
    pltpu.sync_copy(idx_hbm.at[pl.ds(wid * ROWS_PER_W, ROWS_PER_W)], idx_v)

    bias_vecs = tuple(bias_v[pl.ds(16 * k, 16)] for k in range(4))
    bufs = (rows0, rows1)
    sems = (sem0, sem1)

    _fire(w_hbm, idx_v, rows0, sem0, 0)

    def chunk_pair(i, carry):
        for b in range(2):
            g = 2 * i + b
            _fire(w_hbm, idx_v, bufs[1 - b], sems[1 - b], g + 1)
            _drain(w_hbm, bufs[b], sems[b])
            _accumulate(bufs[b], out_v, bias_vecs, g)
        return carry

    lax.fori_loop(0, (N_CHUNKS - 2) // 2, chunk_pair, 0)
    g = N_CHUNKS - 2
    _fire(w_hbm, idx_v, bufs[1], sems[1], g + 1)
    _drain(w_hbm, bufs[0], sems[0])
    _accumulate(bufs[0], out_v, bias_vecs, g)
    _drain(w_hbm, bufs[1], sems[1])
    _accumulate(bufs[1], out_v, bias_vecs, g + 1)

    pltpu.sync_copy(out_v, out_hbm.at[pl.ds(wid * BAGS_PER_W, BAGS_PER_W)])


@jax.jit
def _embed_bag(indices, w, bias):
    mesh = plsc.VectorSubcoreMesh(core_axis_name="c", subcore_axis_name="s")
    run = pl.kernel(
        _sc_body,
        out_type=jax.ShapeDtypeStruct((B, D), jnp.float32),
        mesh=mesh,
        scratch_types=[
            pltpu.VMEM((ROWS_PER_W,), jnp.int32),
            pltpu.VMEM((CHUNK_ROWS, D), jnp.float32),
            pltpu.VMEM((CHUNK_ROWS, D), jnp.float32),
            pltpu.VMEM((BAGS_PER_W, D), jnp.float32),
            pltpu.VMEM((D,), jnp.float32),
            pltpu.SemaphoreType.DMA,
            pltpu.SemaphoreType.DMA,
        ],
        compiler_params=pltpu.CompilerParams(use_tc_tiling_on_sc=False),
    )
    return run(indices, w, bias)


def kernel(indices, offsets, W, bias):
    del offsets
    return _embed_bag(indices.astype(jnp.int32), W, bias)
